# baseline (speedup 1.0000x reference)
import jax
import jax.numpy as jnp
import numpy as np
from jax.experimental import pallas as pl


def _ln_seg(h, c0, n, pad, lw, lb):
    sl = h[:, c0:c0 + n]
    mu = jnp.mean(sl, axis=1, keepdims=True)
    d = sl - mu
    var = jnp.mean(d * d, axis=1, keepdims=True)
    r = jnp.float32(1.0) / jnp.sqrt(var + jnp.float32(1e-5))
    full = h[:, c0:c0 + pad]
    return (full - mu) * r * lw + lb


def _ln_arr(sl, n, lw, lb):
    stat = sl[:, :n]
    mu = jnp.mean(stat, axis=1, keepdims=True)
    d = stat - mu
    var = jnp.mean(d * d, axis=1, keepdims=True)
    r = jnp.float32(1.0) / jnp.sqrt(var + jnp.float32(1e-5))
    return (sl - mu) * r * lw + lb


def _feat_kernel(x_ref, G1_ref, G2a_ref, G2b_ref, G3_ref, b1_ref, b2_ref, b3_ref,
                 S1_ref, SA_ref, SB_ref, S3_ref,
                 lw0_ref, lb0_ref, lw1_ref, lb1_ref, lw2_ref, lb2_ref,
                 lw3_ref, lb3_ref, fcw_ref, fcb_ref, qkw_ref, qkb_ref,
                 out_ref):
    f32 = jnp.float32
    eps = jnp.float32(1e-5)
    dot = lambda a, b: jax.lax.dot(a, b, preferred_element_type=f32)
    h1 = jax.nn.relu(dot(x_ref[...], G1_ref[...]) + b1_ref[...])
    m1 = dot(h1, S1_ref[...])
    v1 = dot(h1 * h1, S1_ref[...]) - m1 * m1
    r1 = jnp.float32(1.0) / jnp.sqrt(v1 + eps)
    segs = [(h1[:, c * 256:(c + 1) * 256] - m1[:, c:c + 1]) * r1[:, c:c + 1]
            * lw0_ref[...] + lb0_ref[...] for c in range(8)]
    winA = jnp.concatenate([s[:, 0:132] for s in segs], axis=1)
    winB = jnp.concatenate([s[:, 128:256] for s in segs], axis=1)
    cA = jax.nn.relu(dot(winA, G2a_ref[...]) + b2_ref[...])
    cB = jax.nn.relu(dot(winB, G2b_ref[...]) + b2_ref[...])
    m2 = dot(cA, SA_ref[...]) + dot(cB, SB_ref[...])
    v2 = dot(cA * cA, SA_ref[...]) + dot(cB * cB, SB_ref[...]) - m2 * m2
    r2 = jnp.float32(1.0) / jnp.sqrt(v2 + eps)
    lw1 = lw1_ref[...]
    lb1 = lb1_ref[...]
    pieces = []
    for co in range(16):
        mu = m2[:, co:co + 1]
        rr = r2[:, co:co + 1]
        pieces.append((cA[:, co * 64: co * 64 + 64] - mu) * rr
                      * lw1[:, :64] + lb1[:, :64])
        pieces.append((cB[:, co * 64: co * 64 + 61] - mu) * rr
                      * lw1[:, 64:] + lb1[:, 64:])
    win3 = jnp.concatenate(pieces, axis=1)
    h3 = jax.nn.relu(dot(win3, G3_ref[...]) + b3_ref[...])
    m3 = dot(h3, S3_ref[...])
    v3 = dot(h3 * h3, S3_ref[...]) - m3 * m3
    r3 = jnp.float32(1.0) / jnp.sqrt(v3 + eps)
    z = fcb_ref[...]
    for co in range(32):
        s3 = ((h3[:, co * 62:(co + 1) * 62] - m3[:, co:co + 1])
              * r3[:, co:co + 1] * lw2_ref[...] + lb2_ref[...])
        z = z + dot(s3, fcw_ref[pl.ds(co * 62, 62), :])
    z = jax.nn.relu(z)
    z = _ln_arr(z, 64, lw3_ref[...], lb3_ref[...])
    out_ref[...] = dot(z, qkw_ref[...]) + qkb_ref[...]


def _features(x2, conv_w0, conv_b0, conv_w1, conv_b1, conv_w2, conv_b2,
              ln_w0, ln_b0, ln_w1, ln_b1, ln_w2, ln_b2,
              fc_w, fc_b, ln_w3, ln_b3, q_w, q_b, k_w, k_b):
    BN = x2.shape[0]
    f32 = jnp.float32
    io512 = jnp.arange(512)[:, None]
    P1 = jnp.stack([(io512 == (2 * jnp.arange(253)[None, :] + kk)).astype(f32)
                    for kk in range(7)])
    G1 = jnp.einsum('ck,kjt->jct', conv_w0[:, 0, :], P1)
    G1 = jnp.pad(G1, ((0, 0), (0, 0), (0, 3))).reshape(512, 2048)
    io132 = jnp.arange(132)[:, None]
    P2a = jnp.stack([(io132 == (2 * jnp.arange(64)[None, :] + kk)).astype(f32)
                     for kk in range(5)])
    G2a = jnp.einsum('oik,kdt->idot', conv_w1, P2a).reshape(1056, 1024)
    io128 = jnp.arange(128)[:, None]
    P2b = jnp.stack([(io128 == (2 * jnp.arange(64)[None, :] + kk)).astype(f32)
                     for kk in range(5)])
    G2b = jnp.einsum('oik,kdt->idot', conv_w1, P2b).reshape(1024, 1024)
    io125 = jnp.arange(125)[:, None]
    P3 = jnp.stack([(io125 == (2 * jnp.arange(62)[None, :] + kk)).astype(f32)
                    for kk in range(3)])
    G3 = jnp.einsum('oik,kdt->idot', conv_w2, P3).reshape(2000, 1984)
    b1 = jnp.broadcast_to(conv_b0[:, None], (8, 256)).reshape(1, 2048)
    b2 = jnp.broadcast_to(conv_b1[:, None], (16, 64)).reshape(1, 1024)
    b3 = jnp.broadcast_to(conv_b2[:, None], (32, 62)).reshape(1, 1984)
    r2048 = jnp.arange(2048)
    S1 = (((r2048[:, None] // 256) == jnp.arange(8)[None, :])
          & ((r2048 % 256) < 253)[:, None]).astype(f32) / 253.0
    r1024 = jnp.arange(1024)
    SA = (((r1024[:, None] // 64) == jnp.arange(16)[None, :])
          ).astype(f32) / 125.0
    SB = (((r1024[:, None] // 64) == jnp.arange(16)[None, :])
          & ((r1024 % 64) < 61)[:, None]).astype(f32) / 125.0
    r1984 = jnp.arange(1984)
    S3 = ((r1984[:, None] // 62) == jnp.arange(32)[None, :]).astype(f32) / 62.0
    lw0 = jnp.pad(ln_w0, (0, 3)).reshape(1, 256)
    lb0 = jnp.pad(ln_b0, (0, 3)).reshape(1, 256)
    qkw = jnp.concatenate([q_w, k_w], axis=1)
    qkb = jnp.concatenate([q_b, k_b]).reshape(1, 128)
    R = 512
    grid = (BN // R,)
    full = lambda shape: pl.BlockSpec(shape, lambda i: tuple(0 for _ in shape))
    out = pl.pallas_call(
        _feat_kernel,
        grid=grid,
        in_specs=[
            pl.BlockSpec((R, 512), lambda i: (i, 0)),
            full((512, 2048)), full((1056, 1024)), full((1024, 1024)),
            full((2000, 1984)),
            full((1, 2048)), full((1, 1024)), full((1, 1984)),
            full((2048, 8)), full((1024, 16)), full((1024, 16)),
            full((1984, 32)),
            full((1, 256)), full((1, 256)),
            full((1, 125)), full((1, 125)),
            full((1, 62)), full((1, 62)),
            full((1, 64)), full((1, 64)),
            full((1984, 64)), full((1, 64)),
            full((64, 128)), full((1, 128)),
        ],
        out_specs=pl.BlockSpec((R, 128), lambda i: (i, 0)),
        out_shape=jax.ShapeDtypeStruct((BN, 128), f32),
    )(x2, G1, G2a, G2b, G3, b1, b2, b3, S1, SA, SB, S3,
      lw0, lb0, ln_w1.reshape(1, 125), ln_b1.reshape(1, 125),
      ln_w2.reshape(1, 62), ln_b2.reshape(1, 62),
      ln_w3.reshape(1, 64), ln_b3.reshape(1, 64),
      fc_w, fc_b.reshape(1, 64), qkw, qkb)
    return out[:, :64], out[:, 64:]


def _count_kernel(idxT_ref, cntT_ref, maskT_ref):
    idxT = idxT_ref[...]
    U, Lc = idxT.shape
    Rb = cntT_ref.shape[0]
    j0 = pl.program_id(0) * Rb
    rowio = jax.lax.broadcasted_iota(jnp.int32, (Rb, Lc), 0) + j0
    cnt = jnp.zeros((Rb, Lc), dtype=jnp.float32)
    for s in range(U):
        cnt = cnt + (rowio == idxT[s:s + 1, :]).astype(jnp.float32)
    cntT_ref[...] = cnt
    maskT_ref[...] = jnp.where(cnt > 0, jnp.float32(0.0), jnp.float32(-3e38))


def _build_count(idxT, L):
    NB = 8
    Rb = L // NB
    return pl.pallas_call(
        _count_kernel,
        grid=(NB,),
        in_specs=[pl.BlockSpec((idxT.shape[0], L), lambda i: (0, 0))],
        out_specs=[pl.BlockSpec((Rb, L), lambda i: (i, 0)),
                   pl.BlockSpec((Rb, L), lambda i: (i, 0))],
        out_shape=[jax.ShapeDtypeStruct((L, L), jnp.float32),
                   jax.ShapeDtypeStruct((L, L), jnp.float32)],
    )(idxT)


def _ck_kernel(cntT_ref, kall_ref, out_ref):
    out_ref[...] = jax.lax.dot_general(
        cntT_ref[...], kall_ref[...], (((0,), (0,)), ((), ())),
        preferred_element_type=jnp.float32)


def _matmul_ck(cnt, kall):
    L, D = kall.shape[0], kall.shape[1]
    return pl.pallas_call(
        _ck_kernel,
        out_shape=jax.ShapeDtypeStruct((L, D), jnp.float32),
    )(cnt, kall)


def _score_topk_kernel(qT_ref, kT_ref, ckT_ref, maskT_ref, out_ref, *, u):
    BH = qT_ref.shape[0]
    N = qT_ref.shape[2]
    maskT = maskT_ref[...]
    rows = []
    for bh in range(BH):
        qT = qT_ref[bh]
        kT = kT_ref[bh]
        qkT = jax.lax.dot_general(kT, qT, (((0,), (0,)), ((), ())),
                                  preferred_element_type=jnp.float32)
        mm = jnp.max(qkT + maskT, axis=0, keepdims=True)
        sums = jnp.sum(qT * ckT_ref[bh], axis=0, keepdims=True)
        rows.append(mm - sums * jnp.float32(1.0 / N))
    M = jnp.concatenate(rows, axis=0)
    iota_row = jax.lax.broadcasted_iota(jnp.int32, (BH, N), 1)
    lane_u = jax.lax.broadcasted_iota(jnp.int32, (1, u), 1)
    mtop = jnp.zeros((BH, u), dtype=jnp.int32)
    for i in range(u):
        mx = jnp.max(M, axis=1, keepdims=True)
        idx = jnp.min(jnp.where(M >= mx, iota_row, jnp.int32(N)),
                      axis=1, keepdims=True)
        mtop = mtop + idx * (lane_u == i).astype(jnp.int32)
        M = jnp.where(iota_row == idx, jnp.float32(-jnp.inf), M)
    out_ref[...] = mtop


def _score_topk(qTh, kTh, ckTh, maskT, u):
    BH, E, N = qTh.shape
    import functools
    return pl.pallas_call(
        functools.partial(_score_topk_kernel, u=u),
        out_shape=jax.ShapeDtypeStruct((BH, u), jnp.int32),
    )(qTh, kTh, ckTh, maskT)


def _attn_scatter_kernel(mtop_ref, q_ref, kT_ref, out_ref):
    H = q_ref.shape[1]
    N = q_ref.shape[2]
    u = mtop_ref.shape[2]
    acc = jnp.zeros((N, N), dtype=jnp.float32)
    row_iota = jax.lax.broadcasted_iota(jnp.int32, (N, u), 0)
    for h in range(H):
        mt = mtop_ref[0, h:h + 1, :]
        oh = (row_iota == mt).astype(jnp.float32)
        qr = jax.lax.dot_general(oh, q_ref[0, h], (((0,), (0,)), ((), ())),
                                 preferred_element_type=jnp.float32)
        qk2 = jax.lax.dot(qr, kT_ref[0, h],
                          preferred_element_type=jnp.float32) * jnp.float32(0.25)
        sm = qk2 - jnp.max(qk2, axis=1, keepdims=True)
        e = jnp.exp(sm)
        attn = e / jnp.sum(e, axis=1, keepdims=True)
        attn = jnp.where(attn < jnp.float32(1.0 / N), jnp.float32(0.0), attn)
        acc = acc + jax.lax.dot(oh, attn, preferred_element_type=jnp.float32)
    out_ref[0] = acc * jnp.float32(1.0 / H)


def _attn_scatter(mtop, q4, kT4):
    B, H, N, E = q4.shape
    u = mtop.shape[2]
    return pl.pallas_call(
        _attn_scatter_kernel,
        grid=(B,),
        in_specs=[
            pl.BlockSpec((1, H, u), lambda b: (b, 0, 0)),
            pl.BlockSpec((1, H, N, E), lambda b: (b, 0, 0, 0)),
            pl.BlockSpec((1, H, E, N), lambda b: (b, 0, 0, 0)),
        ],
        out_specs=pl.BlockSpec((1, N, N), lambda b: (b, 0, 0)),
        out_shape=jax.ShapeDtypeStruct((B, N, N), jnp.float32),
    )(mtop, q4, kT4)


def kernel(x, conv_w0, conv_b0, conv_w1, conv_b1, conv_w2, conv_b2,
           ln_w0, ln_b0, ln_w1, ln_b1, ln_w2, ln_b2,
           fc_w, fc_b, ln_w3, ln_b3, q_w, q_b, k_w, k_b, index_sample):
    B, N, S = x.shape
    H = 4
    factor = 5
    q_flat, k_flat = _features(
        x.reshape(B * N, S), conv_w0, conv_b0, conv_w1, conv_b1,
        conv_w2, conv_b2, ln_w0, ln_b0, ln_w1, ln_b1, ln_w2, ln_b2,
        fc_w, fc_b, ln_w3, ln_b3, q_w, q_b, k_w, k_b)
    L = N
    E = q_flat.shape[1] // H
    logL = int(np.ceil(np.log(L)))
    u = min(factor * logL, L)

    qTh = q_flat.reshape(B, N, H, E).transpose(0, 2, 3, 1).reshape(B * H, E, N)
    kTh = k_flat.reshape(B, N, H, E).transpose(0, 2, 3, 1).reshape(B * H, E, N)
    kall = k_flat.reshape(B, N, H * E).transpose(1, 0, 2).reshape(N, B * H * E)

    cntT, maskT = _build_count(index_sample.T, L)
    ck = _matmul_ck(cntT, kall)
    ckTh = ck.reshape(N, B, H, E).transpose(1, 2, 3, 0).reshape(B * H, E, N)

    mtop = _score_topk(qTh, kTh, ckTh, maskT, u)
    mtop = mtop.reshape(B, H, u)
    q4 = q_flat.reshape(B, N, H, E).transpose(0, 2, 1, 3)
    return _attn_scatter(mtop, q4, kTh.reshape(B, H, E, N))


# speedup vs baseline: 1.5540x; 1.5540x over previous
import jax
import jax.numpy as jnp
import numpy as np
from jax.experimental import pallas as pl


def _layernorm(x, w, b, eps=1e-5):
    mu = x.mean(-1, keepdims=True)
    var = ((x - mu) ** 2).mean(-1, keepdims=True)
    return (x - mu) / jnp.sqrt(var + eps) * w + b


def _conv1d(x, W, b, stride):
    y = jax.lax.conv_general_dilated(x, W, (stride,), 'VALID',
                                     dimension_numbers=('NCH', 'OIH', 'NCH'))
    return y + b[None, :, None]


def _count_kernel(idxT_ref, cntT_ref, maskT_ref):
    idxT = idxT_ref[...]
    U, Lc = idxT.shape
    Rb = cntT_ref.shape[0]
    j0 = pl.program_id(0) * Rb
    rowio = jax.lax.broadcasted_iota(jnp.int32, (Rb, Lc), 0) + j0
    cnt = jnp.zeros((Rb, Lc), dtype=jnp.float32)
    for s in range(U):
        cnt = cnt + (rowio == idxT[s:s + 1, :]).astype(jnp.float32)
    cntT_ref[...] = cnt
    maskT_ref[...] = jnp.where(cnt > 0, jnp.float32(0.0), jnp.float32(-3e38))


def _build_count(idxT, L):
    NB = 8
    Rb = L // NB
    return pl.pallas_call(
        _count_kernel,
        grid=(NB,),
        in_specs=[pl.BlockSpec((idxT.shape[0], L), lambda i: (0, 0))],
        out_specs=[pl.BlockSpec((Rb, L), lambda i: (i, 0)),
                   pl.BlockSpec((Rb, L), lambda i: (i, 0))],
        out_shape=[jax.ShapeDtypeStruct((L, L), jnp.float32),
                   jax.ShapeDtypeStruct((L, L), jnp.float32)],
    )(idxT)


def _ck_kernel(cntT_ref, kall_ref, out_ref):
    out_ref[...] = jax.lax.dot_general(
        cntT_ref[...], kall_ref[...], (((0,), (0,)), ((), ())),
        preferred_element_type=jnp.float32)


def _matmul_ck(cnt, kall):
    L, D = kall.shape[0], kall.shape[1]
    return pl.pallas_call(
        _ck_kernel,
        out_shape=jax.ShapeDtypeStruct((L, D), jnp.float32),
    )(cnt, kall)


def _score_topk_kernel(qT_ref, kT_ref, ckT_ref, maskT_ref, out_ref, *, u):
    BH = qT_ref.shape[0]
    N = qT_ref.shape[2]
    maskT = maskT_ref[...]
    rows = []
    for bh in range(BH):
        qT = qT_ref[bh]
        kT = kT_ref[bh]
        qkT = jax.lax.dot_general(kT, qT, (((0,), (0,)), ((), ())),
                                  preferred_element_type=jnp.float32)
        mm = jnp.max(qkT + maskT, axis=0, keepdims=True)
        sums = jnp.sum(qT * ckT_ref[bh], axis=0, keepdims=True)
        rows.append(mm - sums * jnp.float32(1.0 / N))
    M = jnp.concatenate(rows, axis=0)
    iota_row = jax.lax.broadcasted_iota(jnp.int32, (BH, N), 1)
    lane_u = jax.lax.broadcasted_iota(jnp.int32, (1, u), 1)
    mtop = jnp.zeros((BH, u), dtype=jnp.int32)
    for i in range(u):
        mx = jnp.max(M, axis=1, keepdims=True)
        idx = jnp.min(jnp.where(M >= mx, iota_row, jnp.int32(N)),
                      axis=1, keepdims=True)
        mtop = mtop + idx * (lane_u == i).astype(jnp.int32)
        M = jnp.where(iota_row == idx, jnp.float32(-jnp.inf), M)
    out_ref[...] = mtop


def _score_topk(qTh, kTh, ckTh, maskT, u):
    BH, E, N = qTh.shape
    import functools
    return pl.pallas_call(
        functools.partial(_score_topk_kernel, u=u),
        out_shape=jax.ShapeDtypeStruct((BH, u), jnp.int32),
    )(qTh, kTh, ckTh, maskT)


def _attn_scatter_kernel(mtop_ref, q_ref, kT_ref, out_ref):
    H = q_ref.shape[1]
    N = q_ref.shape[2]
    u = mtop_ref.shape[2]
    acc = jnp.zeros((N, N), dtype=jnp.float32)
    row_iota = jax.lax.broadcasted_iota(jnp.int32, (N, u), 0)
    for h in range(H):
        mt = mtop_ref[0, h:h + 1, :]
        oh = (row_iota == mt).astype(jnp.float32)
        qr = jax.lax.dot_general(oh, q_ref[0, h], (((0,), (0,)), ((), ())),
                                 preferred_element_type=jnp.float32)
        qk2 = jax.lax.dot(qr, kT_ref[0, h],
                          preferred_element_type=jnp.float32) * jnp.float32(0.25)
        sm = qk2 - jnp.max(qk2, axis=1, keepdims=True)
        e = jnp.exp(sm)
        attn = e / jnp.sum(e, axis=1, keepdims=True)
        attn = jnp.where(attn < jnp.float32(1.0 / N), jnp.float32(0.0), attn)
        acc = acc + jax.lax.dot(oh, attn, preferred_element_type=jnp.float32)
    out_ref[0] = acc * jnp.float32(1.0 / H)


def _attn_scatter(mtop, q4, kT4):
    B, H, N, E = q4.shape
    u = mtop.shape[2]
    return pl.pallas_call(
        _attn_scatter_kernel,
        grid=(B,),
        in_specs=[
            pl.BlockSpec((1, H, u), lambda b: (b, 0, 0)),
            pl.BlockSpec((1, H, N, E), lambda b: (b, 0, 0, 0)),
            pl.BlockSpec((1, H, E, N), lambda b: (b, 0, 0, 0)),
        ],
        out_specs=pl.BlockSpec((1, N, N), lambda b: (b, 0, 0)),
        out_shape=jax.ShapeDtypeStruct((B, N, N), jnp.float32),
    )(mtop, q4, kT4)


def kernel(x, conv_w0, conv_b0, conv_w1, conv_b1, conv_w2, conv_b2,
           ln_w0, ln_b0, ln_w1, ln_b1, ln_w2, ln_b2,
           fc_w, fc_b, ln_w3, ln_b3, q_w, q_b, k_w, k_b, index_sample):
    B, N, S = x.shape
    H = 4
    factor = 5
    h = x.reshape(B * N, 1, S)
    layers = [(conv_w0, conv_b0, ln_w0, ln_b0, 2),
              (conv_w1, conv_b1, ln_w1, ln_b1, 2),
              (conv_w2, conv_b2, ln_w2, ln_b2, 2)]
    for (W, b, lw, lb, s) in layers:
        h = _conv1d(h, W, b, s)
        h = jax.nn.relu(h)
        h = _layernorm(h, lw, lb)
    h = h.reshape(B * N, -1)
    h = jax.nn.relu(h @ fc_w + fc_b)
    h = _layernorm(h, ln_w3, ln_b3)
    q_flat = h @ q_w + q_b
    k_flat = h @ k_w + k_b
    L = N
    E = q_flat.shape[1] // H
    logL = int(np.ceil(np.log(L)))
    u = min(factor * logL, L)

    qTh = q_flat.reshape(B, N, H, E).transpose(0, 2, 3, 1).reshape(B * H, E, N)
    kTh = k_flat.reshape(B, N, H, E).transpose(0, 2, 3, 1).reshape(B * H, E, N)
    kall = k_flat.reshape(B, N, H * E).transpose(1, 0, 2).reshape(N, B * H * E)

    cntT, maskT = _build_count(index_sample.T, L)
    ck = _matmul_ck(cntT, kall)
    ckTh = ck.reshape(N, B, H, E).transpose(1, 2, 3, 0).reshape(B * H, E, N)

    mtop = _score_topk(qTh, kTh, ckTh, maskT, u)
    mtop = mtop.reshape(B, H, u)
    q4 = q_flat.reshape(B, N, H, E).transpose(0, 2, 1, 3)
    return _attn_scatter(mtop, q4, kTh.reshape(B, H, E, N))
